# Initial kernel scaffold; baseline (speedup 1.0000x reference)
import jax
import jax.numpy as jnp
from jax.experimental import pallas as pl


def kernel(x, gcn_weight, gcn_bias, gcn_qw, gcn_qb, gcn_kw, gcn_kb, gcn_vw, gcn_vb, qw, qb, kw, kb, vw, vb):
    raise NotImplementedError("write your pallas kernel here")


# speedup vs baseline: 2.0271x; 2.0271x over previous
import functools

import jax
import jax.numpy as jnp
from jax.experimental import pallas as pl
from jax.experimental.pallas import tpu as pltpu

F32 = jnp.float32

GROUP = 32
C = 96
H = 224
W = 224
PIX = 3136
PIX4 = 784
HW = H * W
KK = 21

RB = 448
LBLK = 3584


def _dot(a, b, dims):
    return jax.lax.dot_general(a, b, (dims, ((), ())),
                               precision=jax.lax.Precision.HIGHEST,
                               preferred_element_type=F32)


def _pool_body(x_ref, xp_ref, xpre_ref):
    xb = x_ref[:, 0]
    s = jnp.sum(xb, axis=1)
    p = jax.lax.broadcasted_iota(jnp.int32, (W, W // 4), 0)
    wq = jax.lax.broadcasted_iota(jnp.int32, (W, W // 4), 1)
    B = jnp.where((p // 4) == wq, 1.0 / 16.0, 0.0).astype(F32)
    xp_blk = _dot(s, B, ((1,), (0,)))
    xp_ref[:, 0, 0, :] = xp_blk
    g = jax.lax.broadcasted_iota(jnp.int32, (GROUP, C), 0)
    c = jax.lax.broadcasted_iota(jnp.int32, (GROUP, C), 1)
    M = jnp.where((c // 3) == g, 1.0 / 3.0, 0.0).astype(F32)
    xpre_ref[:, 0, 0, :] = _dot(M, xp_blk, ((1,), (0,)))


def _pool(x4):
    return pl.pallas_call(
        _pool_body,
        grid=(H // 4,),
        in_specs=[pl.BlockSpec((C, 1, 4, W), lambda i: (0, i, 0, 0))],
        out_specs=[
            pl.BlockSpec((C, 1, 1, W // 4), lambda i: (0, i, 0, 0)),
            pl.BlockSpec((GROUP, 1, 1, W // 4), lambda i: (0, i, 0, 0)),
        ],
        out_shape=[
            jax.ShapeDtypeStruct((C, H // 4, 1, W // 4), F32),
            jax.ShapeDtypeStruct((GROUP, H // 4, 1, W // 4), F32),
        ],
    )(x4)


def _adj_body(adj_ref, kw_ref, kb_ref, xp_ref, adjf_ref, k2_ref):
    adj = adj_ref[...]
    jdx = jax.lax.broadcasted_iota(jnp.int32, (GROUP, GROUP), 1)
    rank = jnp.zeros((GROUP, GROUP), jnp.int32)
    for k in range(GROUP):
        col = adj[:, k:k + 1]
        gt = (col > adj).astype(jnp.int32)
        eq = ((col == adj) & (k < jdx)).astype(jnp.int32)
        rank = rank + gt + eq
    keep = rank < KK
    madj = jnp.where(keep, adj, -jnp.inf)
    adjf_ref[...] = jax.nn.softmax(madj, axis=-1)
    k2_ref[...] = _dot(kw_ref[...], xp_ref[...], ((1,), (1,))) + kb_ref[...]


def _adj(adj, kw, kb2, xp):
    return pl.pallas_call(
        _adj_body,
        out_shape=[
            jax.ShapeDtypeStruct((GROUP, GROUP), F32),
            jax.ShapeDtypeStruct((PIX4, C), F32),
        ],
    )(adj, kw, kb2, xp)


def _gcn_body(xpre_ref, adjf_ref, gvw_ref, gvb_ref, gw_ref, gb_ref,
              gout_ref, acc_ref):
    i = pl.program_id(0)

    @pl.when(i == 0)
    def _():
        acc_ref[...] = jnp.zeros_like(acc_ref)

    value_blk = _dot(xpre_ref[...], gvw_ref[...], ((1,), (1,))) + gvb_ref[0]
    acc_ref[...] += _dot(value_blk, gw_ref[...], ((1,), (0,)))

    @pl.when(i == (PIX // RB) - 1)
    def _():
        gout_ref[...] = (_dot(adjf_ref[...], acc_ref[...], ((1,), (0,)))
                         + gb_ref[...])


def _gcn(xpre, adjf, gvw, gvb2, gw, gb2):
    nsteps = PIX // RB
    return pl.pallas_call(
        _gcn_body,
        grid=(nsteps,),
        in_specs=[
            pl.BlockSpec((GROUP, PIX), lambda i: (0, 0)),
            pl.BlockSpec((GROUP, GROUP), lambda i: (0, 0)),
            pl.BlockSpec((RB, PIX), lambda i: (i, 0)),
            pl.BlockSpec((1, 1, RB), lambda i: (i, 0, 0)),
            pl.BlockSpec((RB, PIX), lambda i: (i, 0)),
            pl.BlockSpec((1, PIX), lambda i: (0, 0)),
        ],
        out_specs=pl.BlockSpec((GROUP, PIX), lambda i: (0, 0)),
        out_shape=jax.ShapeDtypeStruct((GROUP, PIX), F32),
        scratch_shapes=[pltpu.VMEM((GROUP, PIX), F32)],
    )(xpre, adjf, gvw, gvb2, gw, gb2)


def _tu_body(gout_ref, k2_ref, qw_ref, qb_ref, vw_ref, vb_ref,
             tu_ref, atten_ref):
    i = pl.program_id(0)

    @pl.when(i == 0)
    def _():
        q2 = _dot(qw_ref[...], gout_ref[...], ((1,), (1,))) + qb_ref[...]
        rawa = _dot(k2_ref[...], q2, ((0,), (0,)))
        atten_ref[...] = jax.nn.softmax(rawa, axis=-1)

    v2t = _dot(gout_ref[...], vw_ref[...], ((1,), (1,))) + vb_ref[0]
    tu_ref[0] = _dot(atten_ref[...], v2t, ((1,), (0,)))


def _tu(gout, k2, qw, qb2, vw, vb2):
    nsteps = PIX // RB
    return pl.pallas_call(
        _tu_body,
        grid=(nsteps,),
        in_specs=[
            pl.BlockSpec((GROUP, PIX), lambda i: (0, 0)),
            pl.BlockSpec((PIX4, C), lambda i: (0, 0)),
            pl.BlockSpec((PIX4, PIX), lambda i: (0, 0)),
            pl.BlockSpec((PIX4, 1), lambda i: (0, 0)),
            pl.BlockSpec((RB, PIX), lambda i: (i, 0)),
            pl.BlockSpec((1, 1, RB), lambda i: (i, 0, 0)),
        ],
        out_specs=pl.BlockSpec((1, C, RB), lambda i: (i, 0, 0)),
        out_shape=jax.ShapeDtypeStruct((PIX // RB, C, RB), F32),
        scratch_shapes=[pltpu.VMEM((C, GROUP), F32)],
    )(gout, k2, qw, qb2, vw, vb2)


def _out_body(tu2_ref, x_ref, out_ref, g_ref):
    i = pl.program_id(0)

    @pl.when(i == 0)
    def _():
        g_ref[...] = _dot(tu2_ref[...], tu2_ref[...], ((1,), (1,)))

    out_ref[...] = _dot(g_ref[...], x_ref[...], ((1,), (0,)))


def _final(tu2, xf):
    nsteps = HW // LBLK
    return pl.pallas_call(
        _out_body,
        grid=(nsteps,),
        in_specs=[
            pl.BlockSpec((C, PIX), lambda i: (0, 0)),
            pl.BlockSpec((C, LBLK), lambda i: (0, i)),
        ],
        out_specs=pl.BlockSpec((C, LBLK), lambda i: (0, i)),
        out_shape=jax.ShapeDtypeStruct((C, HW), F32),
        scratch_shapes=[pltpu.VMEM((C, C), F32)],
    )(tu2, xf)


@jax.jit
def kernel(x, gcn_weight, gcn_bias, gcn_qw, gcn_qb, gcn_kw, gcn_kb,
           gcn_vw, gcn_vb, qw, qb, kw, kb, vw, vb):
    x4 = x.reshape(C, H // 4, 4, W)
    xp3, xpre3 = _pool(x4)
    xp = xp3.reshape(C, PIX)
    xpre = xpre3.reshape(GROUP, PIX)

    xp_m = x.reshape(1, C, H // 4, 4, W // 4, 4).mean(axis=(3, 5))
    xpre_m = xp_m.reshape(1, GROUP, C // GROUP, PIX).mean(axis=2)
    xt_m = jnp.transpose(xpre_m, (0, 2, 1))
    q_m = jnp.einsum('oc,bcl->bol', gcn_qw, xt_m) + gcn_qb[None, :, None]
    k_m = jnp.einsum('oc,bcl->bol', gcn_kw, xt_m) + gcn_kb[None, :, None]
    kt_m = jnp.transpose(k_m, (0, 2, 1))
    adj = jax.nn.softmax(jnp.einsum('bsp,bpt->bst', kt_m, q_m), axis=-1)[0]

    adjf, k2 = _adj(adj, kw, kb.reshape(PIX4, 1), xp)

    gout = _gcn(xpre, adjf, gcn_vw, gcn_vb.reshape(PIX // RB, 1, RB),
                gcn_weight, gcn_bias.reshape(1, PIX))

    tu3 = _tu(gout, k2, qw, qb.reshape(PIX4, 1),
              vw, vb.reshape(PIX // RB, 1, RB))

    tu2 = tu3.transpose(0, 2, 1).reshape(C, PIX)

    out = _final(tu2, x.reshape(C, HW))
    return out.reshape(1, C, H, W)
